# Initial kernel scaffold; baseline (speedup 1.0000x reference)
import jax
import jax.numpy as jnp
from jax.experimental import pallas as pl


def kernel(x_clause, x_literal, x_term, x_symbol, x_variable, params, ei_contains_literal, ei_has_atom, ei_has_arg, ei_symbol_of, ei_var_occurrence, ei_shared_variable):
    raise NotImplementedError("write your pallas kernel here")


# speedup vs baseline: 1.0605x; 1.0605x over previous
import functools

import jax
import jax.numpy as jnp
from jax.experimental import pallas as pl

HID = 256
EMBD = 512
SYMV = 10000
SYMD = 64
NFEAT = {"clause": 7, "literal": 3, "term": 8, "symbol": 6, "variable": 1}
NN = {"clause": 10000, "literal": 30000, "term": 60000, "symbol": 10000, "variable": 15000}
ETYPES = [
    ("contains_literal", False, "clause", "literal", "contains_literal"),
    ("has_atom", False, "literal", "term", "has_atom"),
    ("has_arg", False, "term", "term", "has_arg"),
    ("symbol_of", False, "term", "symbol", "symbol_of"),
    ("var_occurrence", False, "variable", "term", "var_occurrence"),
    ("shared_variable", False, "variable", "variable", "shared_variable"),
    ("contains_literal", True, "literal", "clause", "rev_contains_literal"),
    ("has_atom", True, "term", "literal", "rev_has_atom"),
    ("has_arg", True, "term", "term", "rev_has_arg"),
    ("symbol_of", True, "symbol", "term", "rev_symbol_of"),
    ("var_occurrence", True, "term", "variable", "rev_var_occurrence"),
]
RBLK = 1000


def _proj_body(x_ref, w_ref, b_ref, o_ref):
    o_ref[...] = jax.nn.relu(
        jnp.dot(x_ref[...], w_ref[...], preferred_element_type=jnp.float32)
        + b_ref[...]
    )


def _in_proj(x, w, b):
    n, f = x.shape
    fp = 8
    xp = jnp.pad(x, ((0, 0), (0, fp - f)))
    wp = jnp.pad(w, ((0, fp - f), (0, 0)))
    return pl.pallas_call(
        _proj_body,
        grid=(n // RBLK,),
        in_specs=[
            pl.BlockSpec((RBLK, fp), lambda i: (i, 0)),
            pl.BlockSpec((fp, HID), lambda i: (0, 0)),
            pl.BlockSpec((1, HID), lambda i: (0, 0)),
        ],
        out_specs=pl.BlockSpec((RBLK, HID), lambda i: (i, 0)),
        out_shape=jax.ShapeDtypeStruct((n, HID), jnp.float32),
    )(xp, wp, b.reshape(1, HID))


def _symcomb_body(x_ref, e_ref, w1_ref, w2_ref, b_ref, o_ref):
    o_ref[...] = jax.nn.relu(
        jnp.dot(x_ref[...], w1_ref[...], preferred_element_type=jnp.float32)
        + jnp.dot(e_ref[...], w2_ref[...], preferred_element_type=jnp.float32)
        + b_ref[...]
    )


def _symcomb(x, emb, wc, bc):
    n = x.shape[0]
    return pl.pallas_call(
        _symcomb_body,
        grid=(n // RBLK,),
        in_specs=[
            pl.BlockSpec((RBLK, HID), lambda i: (i, 0)),
            pl.BlockSpec((RBLK, SYMD), lambda i: (i, 0)),
            pl.BlockSpec((HID, HID), lambda i: (0, 0)),
            pl.BlockSpec((SYMD, HID), lambda i: (0, 0)),
            pl.BlockSpec((1, HID), lambda i: (0, 0)),
        ],
        out_specs=pl.BlockSpec((RBLK, HID), lambda i: (i, 0)),
        out_shape=jax.ShapeDtypeStruct((n, HID), jnp.float32),
    )(x, emb, wc[:HID], wc[HID:], bc.reshape(1, HID))


def _update_body(k, x_ref, *refs):
    s_refs = refs[:k]
    c_refs = refs[k : 2 * k]
    wl_ref, wr_ref, bl_ref, g_ref, b_ref, o_ref = refs[2 * k :]
    xb = x_ref[...]
    wr_sum = jnp.sum(wr_ref[...], axis=0)
    o = jnp.dot(xb, wr_sum, preferred_element_type=jnp.float32)
    o = o + jnp.sum(bl_ref[...], axis=0)[None, :]
    for i in range(k):
        ic = 1.0 / jnp.maximum(c_refs[i][...], 1.0)
        o = o + jnp.dot(
            s_refs[i][...] * ic, wl_ref[i], preferred_element_type=jnp.float32
        )
    h = o + xb
    m = jnp.mean(h, axis=-1, keepdims=True)
    v = jnp.mean((h - m) ** 2, axis=-1, keepdims=True)
    o_ref[...] = (h - m) * jax.lax.rsqrt(v + 1e-5) * g_ref[...] + b_ref[...]


def _update(x, s_list, cnt_list, wl, wr, bl, g, b):
    n = x.shape[0]
    k = len(s_list)
    in_specs = [pl.BlockSpec((RBLK, HID), lambda i: (i, 0))]
    in_specs += [pl.BlockSpec((RBLK, HID), lambda i: (i, 0))] * k
    in_specs += [pl.BlockSpec((RBLK, 1), lambda i: (i, 0))] * k
    in_specs += [
        pl.BlockSpec((k, HID, HID), lambda i: (0, 0, 0)),
        pl.BlockSpec((k, HID, HID), lambda i: (0, 0, 0)),
        pl.BlockSpec((k, HID), lambda i: (0, 0)),
        pl.BlockSpec((1, HID), lambda i: (0, 0)),
        pl.BlockSpec((1, HID), lambda i: (0, 0)),
    ]
    return pl.pallas_call(
        functools.partial(_update_body, k),
        grid=(n // RBLK,),
        in_specs=in_specs,
        out_specs=pl.BlockSpec((RBLK, HID), lambda i: (i, 0)),
        out_shape=jax.ShapeDtypeStruct((n, HID), jnp.float32),
    )(x, *s_list, *cnt_list, wl, wr, bl, g.reshape(1, HID), b.reshape(1, HID))


def _head_body(x_ref, w1_ref, b1_ref, w2_ref, b2_ref, o_ref):
    h = jax.nn.relu(
        jnp.dot(x_ref[...], w1_ref[...], preferred_element_type=jnp.float32)
        + b1_ref[...]
    )
    o_ref[...] = (
        jnp.dot(h, w2_ref[...], preferred_element_type=jnp.float32) + b2_ref[...]
    )


def _head(x, w1, b1, w2, b2):
    n = x.shape[0]
    return pl.pallas_call(
        _head_body,
        grid=(n // RBLK,),
        in_specs=[
            pl.BlockSpec((RBLK, HID), lambda i: (i, 0)),
            pl.BlockSpec((HID, HID), lambda i: (0, 0)),
            pl.BlockSpec((1, HID), lambda i: (0, 0)),
            pl.BlockSpec((HID, EMBD), lambda i: (0, 0)),
            pl.BlockSpec((1, EMBD), lambda i: (0, 0)),
        ],
        out_specs=pl.BlockSpec((RBLK, EMBD), lambda i: (i, 0)),
        out_shape=jax.ShapeDtypeStruct((n, EMBD), jnp.float32),
    )(x, w1, b1.reshape(1, HID), w2, b2.reshape(1, EMBD))


def kernel(x_clause, x_literal, x_term, x_symbol, x_variable, params,
           ei_contains_literal, ei_has_atom, ei_has_arg, ei_symbol_of,
           ei_var_occurrence, ei_shared_variable):
    xs = {"clause": x_clause, "literal": x_literal, "term": x_term,
          "symbol": x_symbol, "variable": x_variable}
    eis = {"contains_literal": ei_contains_literal, "has_atom": ei_has_atom,
           "has_arg": ei_has_arg, "symbol_of": ei_symbol_of,
           "var_occurrence": ei_var_occurrence,
           "shared_variable": ei_shared_variable}

    x = {nt: _in_proj(xs[nt], *params["in_proj"][nt]) for nt in NN}
    sym_ids = jnp.clip(xs["symbol"][:, 0].astype(jnp.int32), 0, SYMV - 1)
    emb = jnp.take(params["sym_table"], sym_ids, axis=0)
    x["symbol"] = _symcomb(x["symbol"], emb, *params["sym_comb"])

    edges = []
    for name, swap, snt, dnt, rel in ETYPES:
        ei = eis[name]
        src, dst = (ei[1], ei[0]) if swap else (ei[0], ei[1])
        edges.append((src, dst, snt, dnt, rel))

    cnts = {}
    for i, (src, dst, snt, dnt, rel) in enumerate(edges):
        c = jax.ops.segment_sum(jnp.ones(dst.shape, jnp.float32), dst,
                                num_segments=NN[dnt])
        cnts[rel] = c.reshape(-1, 1)

    for layer in params["layers"]:
        s_by_dst = {nt: [] for nt in NN}
        for src, dst, snt, dnt, rel in edges:
            msg = jnp.take(x[snt], src, axis=0)
            s = jax.ops.segment_sum(msg, dst, num_segments=NN[dnt])
            s_by_dst[dnt].append((rel, s))
        newx = {}
        for nt in NN:
            items = s_by_dst[nt]
            s_list = [s for _, s in items]
            cnt_list = [cnts[rel] for rel, _ in items]
            wl = jnp.stack([layer["convs"][rel][0] for rel, _ in items])
            bl = jnp.stack([layer["convs"][rel][1] for rel, _ in items])
            wr = jnp.stack([layer["convs"][rel][2] for rel, _ in items])
            g, b = layer["norms"][nt]
            newx[nt] = _update(x[nt], s_list, cnt_list, wl, wr, bl, g, b)
        x = newx

    return _head(x["clause"], *params["out1"], *params["out2"])
